# Initial kernel scaffold; baseline (speedup 1.0000x reference)
import jax
import jax.numpy as jnp
from jax.experimental import pallas as pl


def kernel(x, edge_index, edge_type, W1, root1, b1, W2, root2, b2, Wlin, blin):
    raise NotImplementedError("write your pallas kernel here")


# speedup vs baseline: 2.1910x; 2.1910x over previous
import jax
import jax.numpy as jnp
from jax import lax
from jax.experimental import pallas as pl
from jax.experimental.pallas import tpu as pltpu
from jax.experimental.pallas import tpu_sc as plsc

N = 10000
E = 320000
D = 128
R = 8
DOUT = 40

NS = 16
NW = NS
CB = 128
CH = 158
EPT = CB * CH
EPAD = EPT * NW
NP = 10240
NPH = 2560
GROW = NPH
CROWS = NP // 16
NPASS = -(-NP // NPH)

HPT = NPH // NS
HPB = HPT // 2
CPT = CROWS // NS

BN = 400
NB = N // BN


def _zero_vmem(ref, nrows, width):
    z16 = jnp.zeros((16,), jnp.float32)

    def body(i, carry):
        for k in range(width // 16):
            ref[i, pl.ds(k * 16, 16)] = z16
        return carry
    lax.fori_loop(0, nrows, body, 0)


def _sc_mesh():
    return plsc.VectorSubcoreMesh(core_axis_name="c", subcore_axis_name="s",
                                  num_cores=1)


def _edge_agg(s, xw_h, out_h, gv, dv, sv, lidxv, buf, bounce, sem, agg_sh):
    for p in range(NPASS):
        _zero_vmem(bounce, HPB, D)
        for q in range(2):
            pltpu.sync_copy(bounce, agg_sh.at[pl.ds(s * HPT + q * HPB, HPB)])
        plsc.subcore_barrier()

        def chunk(j, carry, p=p):
            pltpu.async_copy(xw_h.at[gv.at[j]], buf, sem).wait()
            j16 = jnp.full((16,), j, jnp.int32)
            for k in range(8):
                d16 = dv[j, pl.ds(k * 16, 16)]
                l16 = d16 - p * NPH
                ok = (l16 >= 0) & (l16 < NPH)
                lidxv[0, pl.ds(k * 16, 16)] = jnp.where(ok, l16, GROW)

            def edge(e, c2):
                sc = plsc.load_gather(sv,
                                      [j16, jnp.full((16,), e, jnp.int32)])
                for k in range(8):
                    buf[e, pl.ds(k * 16, 16)] = buf[e, pl.ds(k * 16, 16)] * sc
                return c2
            lax.fori_loop(0, CB, edge, 0)
            pltpu.sync_copy(buf, agg_sh.at[lidxv.at[0]], add=True)
            return carry
        lax.fori_loop(0, CH, chunk, 0)
        plsc.subcore_barrier()

        for q in range(2):
            pltpu.sync_copy(agg_sh.at[pl.ds(s * HPT + q * HPB, HPB)], bounce)
            pltpu.sync_copy(
                bounce, out_h.at[pl.ds(p * NPH + s * HPT + q * HPB, HPB)])
        plsc.subcore_barrier()


def _agg_body(xw_h, gidx_h, dst_h, et_h, out_h, cnt_out,
              gv, dv, ev, sv, lidxv, buf, bounce, sem, agg_sh):
    s = lax.axis_index("s")
    lanes = lax.iota(jnp.int32, 16)
    ones16 = jnp.full((16,), 1.0, jnp.float32)
    zerof16 = jnp.zeros((16,), jnp.float32)

    pltpu.sync_copy(gidx_h.at[s], gv)
    pltpu.sync_copy(dst_h.at[s], dv)
    pltpu.sync_copy(et_h.at[s], ev)

    _zero_vmem(bounce, CPT, D)
    pltpu.sync_copy(bounce.at[pl.ds(0, CPT)], agg_sh.at[pl.ds(s * CPT, CPT)])
    _zero_vmem(buf, CB, D)
    plsc.subcore_barrier()

    def count_chunk(j, carry):
        for k in range(8):
            d16 = dv[j, pl.ds(k * 16, 16)]
            e16 = ev[j, pl.ds(k * 16, 16)]
            col = (d16 & 15) * 8 + e16
            plsc.store_scatter(buf, [lanes + k * 16, col], ones16)
            lidxv[0, pl.ds(k * 16, 16)] = lax.shift_right_logical(d16, 4)
        pltpu.sync_copy(buf, agg_sh.at[lidxv.at[0]], add=True)
        for k in range(8):
            d16 = dv[j, pl.ds(k * 16, 16)]
            e16 = ev[j, pl.ds(k * 16, 16)]
            col = (d16 & 15) * 8 + e16
            plsc.store_scatter(buf, [lanes + k * 16, col], zerof16)
        return carry
    lax.fori_loop(0, CH, count_chunk, 0)
    plsc.subcore_barrier()

    pltpu.sync_copy(agg_sh.at[pl.ds(s * CPT, CPT)], buf.at[pl.ds(0, CPT)])
    pltpu.sync_copy(buf.at[pl.ds(0, CPT)], cnt_out.at[pl.ds(s * CPT, CPT)])
    plsc.subcore_barrier()

    def scale_chunk(j, carry):
        for k in range(8):
            d16 = dv[j, pl.ds(k * 16, 16)]
            lidxv[0, pl.ds(k * 16, 16)] = lax.shift_right_logical(d16, 4)
        pltpu.sync_copy(cnt_out.at[lidxv.at[0]], buf)
        for k in range(8):
            d16 = dv[j, pl.ds(k * 16, 16)]
            e16 = ev[j, pl.ds(k * 16, 16)]
            cnt16 = plsc.load_gather(buf,
                                     [lanes + k * 16, (d16 & 15) * 8 + e16])
            sv[j, pl.ds(k * 16, 16)] = 1.0 / jnp.maximum(cnt16, 1.0)
        return carry
    lax.fori_loop(0, CH, scale_chunk, 0)

    _edge_agg(s, xw_h, out_h, gv, dv, sv, lidxv, buf, bounce, sem, agg_sh)


def _agg_kernel():
    return pl.kernel(
        _agg_body,
        mesh=_sc_mesh(),
        compiler_params=pltpu.CompilerParams(needs_layout_passes=False),
        out_type=(jax.ShapeDtypeStruct((NP, D), jnp.float32),
                  jax.ShapeDtypeStruct((CROWS, D), jnp.float32)),
        scratch_types=[
            pltpu.VMEM((CH, CB), jnp.int32),
            pltpu.VMEM((CH, CB), jnp.int32),
            pltpu.VMEM((CH, CB), jnp.int32),
            pltpu.VMEM((CH, CB), jnp.float32),
            pltpu.VMEM((1, CB), jnp.int32),
            pltpu.VMEM((CB, D), jnp.float32),
            pltpu.VMEM((HPB, D), jnp.float32),
            pltpu.SemaphoreType.DMA,
            pltpu.VMEM_SHARED((NPH + 8, D), jnp.float32),
        ],
    )


def _xw_body(x_ref, w_ref, o_ref):
    o_ref[0] = jnp.dot(x_ref[...], w_ref[0],
                       preferred_element_type=jnp.float32)


def _per_rel_transform(x, w):
    return pl.pallas_call(
        _xw_body,
        grid=(NB, R),
        in_specs=[
            pl.BlockSpec((BN, D), lambda i, r: (i, 0)),
            pl.BlockSpec((1, D, D), lambda i, r: (r, 0, 0)),
        ],
        out_specs=pl.BlockSpec((1, BN, D), lambda i, r: (r, i, 0)),
        out_shape=jax.ShapeDtypeStruct((R, N, D), jnp.float32),
    )(x, w)


def _h_body(a_ref, x_ref, rt_ref, b_ref, o_ref):
    o_ref[...] = jnp.maximum(
        a_ref[...]
        + jnp.dot(x_ref[...], rt_ref[...], preferred_element_type=jnp.float32)
        + b_ref[...], 0.0)


def _layer_update(a, x, root, b):
    return pl.pallas_call(
        _h_body,
        grid=(NB,),
        in_specs=[
            pl.BlockSpec((BN, D), lambda i: (i, 0)),
            pl.BlockSpec((BN, D), lambda i: (i, 0)),
            pl.BlockSpec((D, D), lambda i: (0, 0)),
            pl.BlockSpec((1, D), lambda i: (0, 0)),
        ],
        out_specs=pl.BlockSpec((BN, D), lambda i: (i, 0)),
        out_shape=jax.ShapeDtypeStruct((N, D), jnp.float32),
    )(a, x, root, b.reshape(1, D))


def _fin_body(h_ref, wl_ref, bl_ref, o_ref):
    logits = jnp.dot(h_ref[...], wl_ref[...],
                     preferred_element_type=jnp.float32) + bl_ref[...]
    m = jnp.max(logits, axis=1, keepdims=True)
    ex = jnp.exp(logits - m)
    lse = jnp.log(jnp.sum(ex, axis=1, keepdims=True)) + m
    o_ref[...] = logits - lse


def _final_linear(h, wlin, blin):
    return pl.pallas_call(
        _fin_body,
        grid=(NB,),
        in_specs=[
            pl.BlockSpec((BN, D), lambda i: (i, 0)),
            pl.BlockSpec((D, DOUT), lambda i: (0, 0)),
            pl.BlockSpec((1, DOUT), lambda i: (0, 0)),
        ],
        out_specs=pl.BlockSpec((BN, DOUT), lambda i: (i, 0)),
        out_shape=jax.ShapeDtypeStruct((N, DOUT), jnp.float32),
    )(h, wlin, blin.reshape(1, DOUT))


def kernel(x, edge_index, edge_type, W1, root1, b1, W2, root2, b2, Wlin, blin):
    src = edge_index[0].astype(jnp.int32)
    dst = edge_index[1].astype(jnp.int32)
    et = edge_type.astype(jnp.int32)

    pad = EPAD - E
    gidx = et * N + src
    gidx_h = jnp.concatenate(
        [gidx, jnp.zeros((pad,), jnp.int32)]).reshape(NW, CH, CB)
    dst_h = jnp.concatenate(
        [dst, jnp.full((pad,), N, jnp.int32)]).reshape(NW, CH, CB)
    et_h = jnp.concatenate(
        [et, jnp.zeros((pad,), jnp.int32)]).reshape(NW, CH, CB)

    Ws = jnp.stack([W1, W2])
    roots = jnp.stack([root1, root2])
    bs = jnp.stack([b1, b2])

    def layer(h, wrb):
        W, root, b = wrb
        xw = _per_rel_transform(h, W).reshape(R * N, D)
        agg, _ = _agg_kernel()(xw, gidx_h, dst_h, et_h)
        return _layer_update(agg[:N], h, root, b), 0

    h2, _ = lax.scan(layer, x, (Ws, roots, bs))
    return _final_linear(h2, Wlin, blin)


# speedup vs baseline: 3.0200x; 1.3783x over previous
import jax
import jax.numpy as jnp
from jax import lax
from jax.experimental import pallas as pl
from jax.experimental.pallas import tpu as pltpu
from jax.experimental.pallas import tpu_sc as plsc

N = 10000
E = 320000
D = 128
R = 8
DOUT = 40

NS = 16
NW = NS
CB = 128
CH = 158
EPT = CB * CH
EPAD = EPT * NW
NP = 10240
NPH = 2560
GROW = NPH
CROWS = NP // 16
NPASS = -(-NP // NPH)

HPT = NPH // NS
HPB = HPT // 2
CPT = CROWS // NS

BN = 400
NB = N // BN


def _zero_vmem(ref, nrows, width):
    z16 = jnp.zeros((16,), jnp.float32)

    def body(i, carry):
        for k in range(width // 16):
            ref[i, pl.ds(k * 16, 16)] = z16
        return carry
    lax.fori_loop(0, nrows, body, 0)


def _sc_mesh():
    return plsc.VectorSubcoreMesh(core_axis_name="c", subcore_axis_name="s",
                                  num_cores=1)


def _mk_lidx(pv, j, lidx, p):
    for k in range(8):
        d16 = pv[j, pl.ds(k * 16, 16)] & 16383
        l16 = d16 - p * NPH
        ok = (l16 >= 0) & (l16 < NPH)
        lidx[0, pl.ds(k * 16, 16)] = jnp.where(ok, l16, GROW)


def _mk_gix(pv, j, gix):
    for k in range(8):
        gix[0, pl.ds(k * 16, 16)] = lax.shift_right_logical(
            pv[j, pl.ds(k * 16, 16)], 14)


def _scale_rows(bufc, sv, j):
    j16 = jnp.full((16,), j, jnp.int32)

    def edge2(e, c2):
        e0 = 2 * e
        sc0 = plsc.load_gather(sv, [j16, jnp.full((16,), e0, jnp.int32)])
        sc1 = plsc.load_gather(sv, [j16, jnp.full((16,), e0 + 1, jnp.int32)])
        for k in range(8):
            bufc[e0, pl.ds(k * 16, 16)] = bufc[e0, pl.ds(k * 16, 16)] * sc0
        for k in range(8):
            bufc[e0 + 1, pl.ds(k * 16, 16)] = (
                bufc[e0 + 1, pl.ds(k * 16, 16)] * sc1)
        return c2
    lax.fori_loop(0, CB // 2, edge2, 0)


def _edge_agg(s, xw_h, out_h, pv, sv, gix0, gix1, l0, l1, b0, b1, bounce,
              sem, agg_sh):
    for p in range(NPASS):
        _zero_vmem(bounce, HPB, D)
        for q in range(2):
            pltpu.sync_copy(bounce, agg_sh.at[pl.ds(s * HPT + q * HPB, HPB)])
        plsc.subcore_barrier()

        _mk_gix(pv, 0, gix0)
        pltpu.async_copy(xw_h.at[gix0.at[0]], b0, sem)

        def pair(t, carry, p=p):
            ja = 2 * t
            jb = 2 * t + 1
            pltpu.make_async_copy(xw_h.at[gix0.at[0]], b0, sem).wait()
            _mk_gix(pv, jb, gix1)
            pltpu.async_copy(xw_h.at[gix1.at[0]], b1, sem)
            _scale_rows(b0, sv, ja)
            _mk_lidx(pv, ja, l0, p)
            pltpu.sync_copy(b0, agg_sh.at[l0.at[0]], add=True)
            pltpu.make_async_copy(xw_h.at[gix1.at[0]], b1, sem).wait()
            jn = jnp.minimum(jb + 1, CH - 1)
            _mk_gix(pv, jn, gix0)
            pltpu.async_copy(xw_h.at[gix0.at[0]], b0, sem)
            _scale_rows(b1, sv, jb)
            _mk_lidx(pv, jb, l1, p)
            pltpu.sync_copy(b1, agg_sh.at[l1.at[0]], add=True)
            return carry
        lax.fori_loop(0, CH // 2, pair, 0)
        pltpu.make_async_copy(xw_h.at[gix0.at[0]], b0, sem).wait()
        plsc.subcore_barrier()

        for q in range(2):
            pltpu.sync_copy(agg_sh.at[pl.ds(s * HPT + q * HPB, HPB)], bounce)
            pltpu.sync_copy(
                bounce, out_h.at[pl.ds(p * NPH + s * HPT + q * HPB, HPB)])
        plsc.subcore_barrier()


def _agg_body(xw_h, pv_h, et_h, out_h, cnt_out,
              pv, ev, sv, gix0, gix1, l0, l1, b0, b1, bounce, sem, agg_sh):
    s = lax.axis_index("s")
    lanes = lax.iota(jnp.int32, 16)
    ones16 = jnp.full((16,), 1.0, jnp.float32)
    zerof16 = jnp.zeros((16,), jnp.float32)

    pltpu.sync_copy(pv_h.at[s], pv)
    pltpu.sync_copy(et_h.at[s], ev)

    _zero_vmem(bounce, CPT, D)
    pltpu.sync_copy(bounce.at[pl.ds(0, CPT)], agg_sh.at[pl.ds(s * CPT, CPT)])
    _zero_vmem(b0, CB, D)
    plsc.subcore_barrier()

    def count_chunk(j, carry):
        for k in range(8):
            d16 = pv[j, pl.ds(k * 16, 16)] & 16383
            e16 = ev[j, pl.ds(k * 16, 16)]
            col = (d16 & 15) * 8 + e16
            plsc.store_scatter(b0, [lanes + k * 16, col], ones16)
            l0[0, pl.ds(k * 16, 16)] = lax.shift_right_logical(d16, 4)
        pltpu.sync_copy(b0, agg_sh.at[l0.at[0]], add=True)
        for k in range(8):
            d16 = pv[j, pl.ds(k * 16, 16)] & 16383
            e16 = ev[j, pl.ds(k * 16, 16)]
            col = (d16 & 15) * 8 + e16
            plsc.store_scatter(b0, [lanes + k * 16, col], zerof16)
        return carry
    lax.fori_loop(0, CH, count_chunk, 0)
    plsc.subcore_barrier()

    pltpu.sync_copy(agg_sh.at[pl.ds(s * CPT, CPT)], b0.at[pl.ds(0, CPT)])
    pltpu.sync_copy(b0.at[pl.ds(0, CPT)], cnt_out.at[pl.ds(s * CPT, CPT)])
    plsc.subcore_barrier()

    def scale_chunk(j, carry):
        for k in range(8):
            d16 = pv[j, pl.ds(k * 16, 16)] & 16383
            l0[0, pl.ds(k * 16, 16)] = lax.shift_right_logical(d16, 4)
        pltpu.sync_copy(cnt_out.at[l0.at[0]], b0)
        for k in range(8):
            d16 = pv[j, pl.ds(k * 16, 16)] & 16383
            e16 = ev[j, pl.ds(k * 16, 16)]
            cnt16 = plsc.load_gather(b0,
                                     [lanes + k * 16, (d16 & 15) * 8 + e16])
            sv[j, pl.ds(k * 16, 16)] = 1.0 / jnp.maximum(cnt16, 1.0)
        return carry
    lax.fori_loop(0, CH, scale_chunk, 0)

    _edge_agg(s, xw_h, out_h, pv, sv, gix0, gix1, l0, l1, b0, b1, bounce,
              sem, agg_sh)


def _agg_kernel():
    return pl.kernel(
        _agg_body,
        mesh=_sc_mesh(),
        compiler_params=pltpu.CompilerParams(needs_layout_passes=False),
        out_type=(jax.ShapeDtypeStruct((NP, D), jnp.float32),
                  jax.ShapeDtypeStruct((CROWS, D), jnp.float32)),
        scratch_types=[
            pltpu.VMEM((CH, CB), jnp.int32),
            pltpu.VMEM((CH, CB), jnp.int32),
            pltpu.VMEM((CH, CB), jnp.float32),
            pltpu.VMEM((1, CB), jnp.int32),
            pltpu.VMEM((1, CB), jnp.int32),
            pltpu.VMEM((1, CB), jnp.int32),
            pltpu.VMEM((1, CB), jnp.int32),
            pltpu.VMEM((CB, D), jnp.float32),
            pltpu.VMEM((CB, D), jnp.float32),
            pltpu.VMEM((HPB, D), jnp.float32),
            pltpu.SemaphoreType.DMA,
            pltpu.VMEM_SHARED((NPH + 8, D), jnp.float32),
        ],
    )


def _xw_body(x_ref, w_ref, o_ref):
    o_ref[0] = jnp.dot(x_ref[...], w_ref[0],
                       preferred_element_type=jnp.float32)


def _per_rel_transform(x, w):
    return pl.pallas_call(
        _xw_body,
        grid=(NB, R),
        in_specs=[
            pl.BlockSpec((BN, D), lambda i, r: (i, 0)),
            pl.BlockSpec((1, D, D), lambda i, r: (r, 0, 0)),
        ],
        out_specs=pl.BlockSpec((1, BN, D), lambda i, r: (r, i, 0)),
        out_shape=jax.ShapeDtypeStruct((R, N, D), jnp.float32),
    )(x, w)


def _h_body(a_ref, x_ref, rt_ref, b_ref, o_ref):
    o_ref[...] = jnp.maximum(
        a_ref[...]
        + jnp.dot(x_ref[...], rt_ref[...], preferred_element_type=jnp.float32)
        + b_ref[...], 0.0)


def _layer_update(a, x, root, b):
    return pl.pallas_call(
        _h_body,
        grid=(NB,),
        in_specs=[
            pl.BlockSpec((BN, D), lambda i: (i, 0)),
            pl.BlockSpec((BN, D), lambda i: (i, 0)),
            pl.BlockSpec((D, D), lambda i: (0, 0)),
            pl.BlockSpec((1, D), lambda i: (0, 0)),
        ],
        out_specs=pl.BlockSpec((BN, D), lambda i: (i, 0)),
        out_shape=jax.ShapeDtypeStruct((N, D), jnp.float32),
    )(a, x, root, b.reshape(1, D))


def _fin_body(h_ref, wl_ref, bl_ref, o_ref):
    logits = jnp.dot(h_ref[...], wl_ref[...],
                     preferred_element_type=jnp.float32) + bl_ref[...]
    m = jnp.max(logits, axis=1, keepdims=True)
    ex = jnp.exp(logits - m)
    lse = jnp.log(jnp.sum(ex, axis=1, keepdims=True)) + m
    o_ref[...] = logits - lse


def _final_linear(h, wlin, blin):
    return pl.pallas_call(
        _fin_body,
        grid=(NB,),
        in_specs=[
            pl.BlockSpec((BN, D), lambda i: (i, 0)),
            pl.BlockSpec((D, DOUT), lambda i: (0, 0)),
            pl.BlockSpec((1, DOUT), lambda i: (0, 0)),
        ],
        out_specs=pl.BlockSpec((BN, DOUT), lambda i: (i, 0)),
        out_shape=jax.ShapeDtypeStruct((N, DOUT), jnp.float32),
    )(h, wlin, blin.reshape(1, DOUT))


def kernel(x, edge_index, edge_type, W1, root1, b1, W2, root2, b2, Wlin, blin):
    src = edge_index[0].astype(jnp.int32)
    dst = edge_index[1].astype(jnp.int32)
    et = edge_type.astype(jnp.int32)

    pad = EPAD - E
    gidx = et * N + src
    pv = gidx * 16384 + dst
    pv_h = jnp.concatenate(
        [pv, jnp.full((pad,), N, jnp.int32)]).reshape(NW, CH, CB)
    et_h = jnp.concatenate(
        [et, jnp.zeros((pad,), jnp.int32)]).reshape(NW, CH, CB)

    Ws = jnp.stack([W1, W2])
    roots = jnp.stack([root1, root2])
    bs = jnp.stack([b1, b2])

    def layer(h, wrb):
        W, root, b = wrb
        xw = _per_rel_transform(h, W).reshape(R * N, D)
        agg, _ = _agg_kernel()(xw, pv_h, et_h)
        return _layer_update(agg[:N], h, root, b), 0

    h2, _ = lax.scan(layer, x, (Ws, roots, bs))
    return _final_linear(h2, Wlin, blin)


# speedup vs baseline: 3.0203x; 1.0001x over previous
import jax
import jax.numpy as jnp
from jax import lax
from jax.experimental import pallas as pl
from jax.experimental.pallas import tpu as pltpu
from jax.experimental.pallas import tpu_sc as plsc

N = 10000
E = 320000
D = 128
R = 8
DOUT = 40

NS = 16
NW = NS
CB = 128
CH = 158
EPT = CB * CH
EPAD = EPT * NW
NP = 10240
NPH = 2560
GROW = NPH
CROWS = NP // 16
NPASS = -(-NP // NPH)

HPT = NPH // NS
HPB = HPT // 2
CPT = CROWS // NS

BN = 400
NB = N // BN


def _zero_vmem(ref, nrows, width):
    z16 = jnp.zeros((16,), jnp.float32)

    def body(i, carry):
        for k in range(width // 16):
            ref[i, pl.ds(k * 16, 16)] = z16
        return carry
    lax.fori_loop(0, nrows, body, 0)


def _sc_mesh():
    return plsc.VectorSubcoreMesh(core_axis_name="c", subcore_axis_name="s",
                                  num_cores=1)


def _mk_lidx(pv, j, lidx, p):
    for k in range(8):
        d16 = pv[j, pl.ds(k * 16, 16)] & 16383
        l16 = d16 - p * NPH
        ok = (l16 >= 0) & (l16 < NPH)
        lidx[0, pl.ds(k * 16, 16)] = jnp.where(ok, l16, GROW)


def _mk_gix(pv, j, gix):
    for k in range(8):
        gix[0, pl.ds(k * 16, 16)] = lax.shift_right_logical(
            pv[j, pl.ds(k * 16, 16)], 14)


def _scale_rows(bufc, sv, j):
    j16 = jnp.full((16,), j, jnp.int32)

    def edge2(e, c2):
        e0 = 2 * e
        sc0 = plsc.load_gather(sv, [j16, jnp.full((16,), e0, jnp.int32)])
        sc1 = plsc.load_gather(sv, [j16, jnp.full((16,), e0 + 1, jnp.int32)])
        for k in range(8):
            bufc[e0, pl.ds(k * 16, 16)] = bufc[e0, pl.ds(k * 16, 16)] * sc0
        for k in range(8):
            bufc[e0 + 1, pl.ds(k * 16, 16)] = (
                bufc[e0 + 1, pl.ds(k * 16, 16)] * sc1)
        return c2
    lax.fori_loop(0, CB // 2, edge2, 0)


def _edge_agg(s, xw_h, out_h, pv, sv, gix0, gix1, l0, l1, b0, b1, bounce,
              sem, sem1, agg_sh):
    for p in range(NPASS):
        _zero_vmem(bounce, HPB, D)
        for q in range(2):
            pltpu.sync_copy(bounce, agg_sh.at[pl.ds(s * HPT + q * HPB, HPB)])
        plsc.subcore_barrier()

        _mk_gix(pv, 0, gix0)
        pltpu.async_copy(xw_h.at[gix0.at[0]], b0, sem)

        def pair(t, carry, p=p):
            ja = 2 * t
            jb = 2 * t + 1
            pltpu.make_async_copy(xw_h.at[gix0.at[0]], b0, sem).wait()
            _mk_gix(pv, jb, gix1)
            pltpu.async_copy(xw_h.at[gix1.at[0]], b1, sem1)
            _scale_rows(b0, sv, ja)
            _mk_lidx(pv, ja, l0, p)
            pltpu.sync_copy(b0, agg_sh.at[l0.at[0]], add=True)
            pltpu.make_async_copy(xw_h.at[gix1.at[0]], b1, sem1).wait()
            jn = jnp.minimum(jb + 1, CH - 1)
            _mk_gix(pv, jn, gix0)
            pltpu.async_copy(xw_h.at[gix0.at[0]], b0, sem)
            _scale_rows(b1, sv, jb)
            _mk_lidx(pv, jb, l1, p)
            pltpu.sync_copy(b1, agg_sh.at[l1.at[0]], add=True)
            return carry
        lax.fori_loop(0, CH // 2, pair, 0)
        pltpu.make_async_copy(xw_h.at[gix0.at[0]], b0, sem).wait()
        plsc.subcore_barrier()

        for q in range(2):
            pltpu.sync_copy(agg_sh.at[pl.ds(s * HPT + q * HPB, HPB)], bounce)
            pltpu.sync_copy(
                bounce, out_h.at[pl.ds(p * NPH + s * HPT + q * HPB, HPB)])
        plsc.subcore_barrier()


def _agg_body(xw_h, pv_h, et_h, out_h, cnt_out,
              pv, ev, sv, gix0, gix1, l0, l1, b0, b1, bounce, sem, sem1,
              agg_sh):
    s = lax.axis_index("s")
    lanes = lax.iota(jnp.int32, 16)
    ones16 = jnp.full((16,), 1.0, jnp.float32)
    zerof16 = jnp.zeros((16,), jnp.float32)

    pltpu.sync_copy(pv_h.at[s], pv)
    pltpu.sync_copy(et_h.at[s], ev)

    _zero_vmem(bounce, CPT, D)
    pltpu.sync_copy(bounce.at[pl.ds(0, CPT)], agg_sh.at[pl.ds(s * CPT, CPT)])
    _zero_vmem(b0, CB, D)
    plsc.subcore_barrier()

    def count_chunk(j, carry):
        for k in range(8):
            d16 = pv[j, pl.ds(k * 16, 16)] & 16383
            e16 = ev[j, pl.ds(k * 16, 16)]
            col = (d16 & 15) * 8 + e16
            plsc.store_scatter(b0, [lanes + k * 16, col], ones16)
            l0[0, pl.ds(k * 16, 16)] = lax.shift_right_logical(d16, 4)
        pltpu.sync_copy(b0, agg_sh.at[l0.at[0]], add=True)
        for k in range(8):
            d16 = pv[j, pl.ds(k * 16, 16)] & 16383
            e16 = ev[j, pl.ds(k * 16, 16)]
            col = (d16 & 15) * 8 + e16
            plsc.store_scatter(b0, [lanes + k * 16, col], zerof16)
        return carry
    lax.fori_loop(0, CH, count_chunk, 0)
    plsc.subcore_barrier()

    pltpu.sync_copy(agg_sh.at[pl.ds(s * CPT, CPT)], b0.at[pl.ds(0, CPT)])
    pltpu.sync_copy(b0.at[pl.ds(0, CPT)], cnt_out.at[pl.ds(s * CPT, CPT)])
    plsc.subcore_barrier()

    def scale_chunk(j, carry):
        for k in range(8):
            d16 = pv[j, pl.ds(k * 16, 16)] & 16383
            l0[0, pl.ds(k * 16, 16)] = lax.shift_right_logical(d16, 4)
        pltpu.sync_copy(cnt_out.at[l0.at[0]], b0)
        for k in range(8):
            d16 = pv[j, pl.ds(k * 16, 16)] & 16383
            e16 = ev[j, pl.ds(k * 16, 16)]
            cnt16 = plsc.load_gather(b0,
                                     [lanes + k * 16, (d16 & 15) * 8 + e16])
            sv[j, pl.ds(k * 16, 16)] = 1.0 / jnp.maximum(cnt16, 1.0)
        return carry
    lax.fori_loop(0, CH, scale_chunk, 0)

    _edge_agg(s, xw_h, out_h, pv, sv, gix0, gix1, l0, l1, b0, b1, bounce,
              sem, sem1, agg_sh)


def _agg_kernel():
    return pl.kernel(
        _agg_body,
        mesh=_sc_mesh(),
        compiler_params=pltpu.CompilerParams(needs_layout_passes=False),
        out_type=(jax.ShapeDtypeStruct((NP, D), jnp.float32),
                  jax.ShapeDtypeStruct((CROWS, D), jnp.float32)),
        scratch_types=[
            pltpu.VMEM((CH, CB), jnp.int32),
            pltpu.VMEM((CH, CB), jnp.int32),
            pltpu.VMEM((CH, CB), jnp.float32),
            pltpu.VMEM((1, CB), jnp.int32),
            pltpu.VMEM((1, CB), jnp.int32),
            pltpu.VMEM((1, CB), jnp.int32),
            pltpu.VMEM((1, CB), jnp.int32),
            pltpu.VMEM((CB, D), jnp.float32),
            pltpu.VMEM((CB, D), jnp.float32),
            pltpu.VMEM((HPB, D), jnp.float32),
            pltpu.SemaphoreType.DMA,
            pltpu.SemaphoreType.DMA,
            pltpu.VMEM_SHARED((NPH + 8, D), jnp.float32),
        ],
    )


def _xw_body(x_ref, w_ref, o_ref):
    o_ref[0] = jnp.dot(x_ref[...], w_ref[0],
                       preferred_element_type=jnp.float32)


def _per_rel_transform(x, w):
    return pl.pallas_call(
        _xw_body,
        grid=(NB, R),
        in_specs=[
            pl.BlockSpec((BN, D), lambda i, r: (i, 0)),
            pl.BlockSpec((1, D, D), lambda i, r: (r, 0, 0)),
        ],
        out_specs=pl.BlockSpec((1, BN, D), lambda i, r: (r, i, 0)),
        out_shape=jax.ShapeDtypeStruct((R, N, D), jnp.float32),
    )(x, w)


def _h_body(a_ref, x_ref, rt_ref, b_ref, o_ref):
    o_ref[...] = jnp.maximum(
        a_ref[...]
        + jnp.dot(x_ref[...], rt_ref[...], preferred_element_type=jnp.float32)
        + b_ref[...], 0.0)


def _layer_update(a, x, root, b):
    return pl.pallas_call(
        _h_body,
        grid=(NB,),
        in_specs=[
            pl.BlockSpec((BN, D), lambda i: (i, 0)),
            pl.BlockSpec((BN, D), lambda i: (i, 0)),
            pl.BlockSpec((D, D), lambda i: (0, 0)),
            pl.BlockSpec((1, D), lambda i: (0, 0)),
        ],
        out_specs=pl.BlockSpec((BN, D), lambda i: (i, 0)),
        out_shape=jax.ShapeDtypeStruct((N, D), jnp.float32),
    )(a, x, root, b.reshape(1, D))


def _fin_body(h_ref, wl_ref, bl_ref, o_ref):
    logits = jnp.dot(h_ref[...], wl_ref[...],
                     preferred_element_type=jnp.float32) + bl_ref[...]
    m = jnp.max(logits, axis=1, keepdims=True)
    ex = jnp.exp(logits - m)
    lse = jnp.log(jnp.sum(ex, axis=1, keepdims=True)) + m
    o_ref[...] = logits - lse


def _final_linear(h, wlin, blin):
    return pl.pallas_call(
        _fin_body,
        grid=(NB,),
        in_specs=[
            pl.BlockSpec((BN, D), lambda i: (i, 0)),
            pl.BlockSpec((D, DOUT), lambda i: (0, 0)),
            pl.BlockSpec((1, DOUT), lambda i: (0, 0)),
        ],
        out_specs=pl.BlockSpec((BN, DOUT), lambda i: (i, 0)),
        out_shape=jax.ShapeDtypeStruct((N, DOUT), jnp.float32),
    )(h, wlin, blin.reshape(1, DOUT))


def kernel(x, edge_index, edge_type, W1, root1, b1, W2, root2, b2, Wlin, blin):
    src = edge_index[0].astype(jnp.int32)
    dst = edge_index[1].astype(jnp.int32)
    et = edge_type.astype(jnp.int32)

    pad = EPAD - E
    gidx = et * N + src
    pv = gidx * 16384 + dst
    pv_h = jnp.concatenate(
        [pv, jnp.full((pad,), N, jnp.int32)]).reshape(NW, CH, CB)
    et_h = jnp.concatenate(
        [et, jnp.zeros((pad,), jnp.int32)]).reshape(NW, CH, CB)

    Ws = jnp.stack([W1, W2])
    roots = jnp.stack([root1, root2])
    bs = jnp.stack([b1, b2])

    def layer(h, wrb):
        W, root, b = wrb
        xw = _per_rel_transform(h, W).reshape(R * N, D)
        agg, _ = _agg_kernel()(xw, pv_h, et_h)
        return _layer_update(agg[:N], h, root, b), 0

    h2, _ = lax.scan(layer, x, (Ws, roots, bs))
    return _final_linear(h2, Wlin, blin)


# speedup vs baseline: 3.0722x; 1.0172x over previous
import jax
import jax.numpy as jnp
from jax import lax
from jax.experimental import pallas as pl
from jax.experimental.pallas import tpu as pltpu
from jax.experimental.pallas import tpu_sc as plsc

N = 10000
E = 320000
D = 128
R = 8
DOUT = 40

NS = 16
NW = NS
CB = 128
CH = 158
EPT = CB * CH
EPAD = EPT * NW
NP = 10240
NPH = 2560
GROW = NPH
CROWS = NP // 16
NPASS = -(-NP // NPH)

HPT = NPH // NS
HPB = HPT // 2
CPT = CROWS // NS

BN = 400
NB = N // BN


def _zero_vmem(ref, nrows, width):
    z16 = jnp.zeros((16,), jnp.float32)

    def body(i, carry):
        for k in range(width // 16):
            ref[i, pl.ds(k * 16, 16)] = z16
        return carry
    lax.fori_loop(0, nrows, body, 0)


def _sc_mesh():
    return plsc.VectorSubcoreMesh(core_axis_name="c", subcore_axis_name="s",
                                  num_cores=1)


def _mk_lidx(pv, j, lidx, p):
    for k in range(8):
        d16 = pv[j, pl.ds(k * 16, 16)] & 16383
        l16 = d16 - p * NPH
        ok = (l16 >= 0) & (l16 < NPH)
        lidx[0, pl.ds(k * 16, 16)] = jnp.where(ok, l16, GROW)


def _mk_gix(pv, j, gix):
    for k in range(8):
        gix[0, pl.ds(k * 16, 16)] = lax.shift_right_logical(
            pv[j, pl.ds(k * 16, 16)], 14)


def _scale_rows(bufc, sv, j):
    j16 = jnp.full((16,), j, jnp.int32)

    def edge4(e, c2):
        e0 = 4 * e
        scs = [plsc.load_gather(sv, [j16, jnp.full((16,), e0 + i, jnp.int32)])
               for i in range(4)]
        for i in range(4):
            for k in range(8):
                bufc[e0 + i, pl.ds(k * 16, 16)] = (
                    bufc[e0 + i, pl.ds(k * 16, 16)] * scs[i])
        return c2
    lax.fori_loop(0, CB // 4, edge4, 0)


def _edge_agg(s, xw_h, out_h, pv, sv, gix0, gix1, l0, l1, b0, b1, bounce,
              sem, sem1, agg_sh):
    for p in range(NPASS):
        _zero_vmem(bounce, HPB, D)
        for q in range(2):
            pltpu.sync_copy(bounce, agg_sh.at[pl.ds(s * HPT + q * HPB, HPB)])
        plsc.subcore_barrier()

        _mk_gix(pv, 0, gix0)
        pltpu.async_copy(xw_h.at[gix0.at[0]], b0, sem)

        def pair(t, carry, p=p):
            ja = 2 * t
            jb = 2 * t + 1
            pltpu.make_async_copy(xw_h.at[gix0.at[0]], b0, sem).wait()
            _mk_gix(pv, jb, gix1)
            pltpu.async_copy(xw_h.at[gix1.at[0]], b1, sem1)
            _scale_rows(b0, sv, ja)
            _mk_lidx(pv, ja, l0, p)
            pltpu.sync_copy(b0, agg_sh.at[l0.at[0]], add=True)
            pltpu.make_async_copy(xw_h.at[gix1.at[0]], b1, sem1).wait()
            jn = jnp.minimum(jb + 1, CH - 1)
            _mk_gix(pv, jn, gix0)
            pltpu.async_copy(xw_h.at[gix0.at[0]], b0, sem)
            _scale_rows(b1, sv, jb)
            _mk_lidx(pv, jb, l1, p)
            pltpu.sync_copy(b1, agg_sh.at[l1.at[0]], add=True)
            return carry
        lax.fori_loop(0, CH // 2, pair, 0)
        pltpu.make_async_copy(xw_h.at[gix0.at[0]], b0, sem).wait()
        plsc.subcore_barrier()

        for q in range(2):
            pltpu.sync_copy(agg_sh.at[pl.ds(s * HPT + q * HPB, HPB)], bounce)
            pltpu.sync_copy(
                bounce, out_h.at[pl.ds(p * NPH + s * HPT + q * HPB, HPB)])
        plsc.subcore_barrier()


def _agg_body(xw_h, pv_h, et_h, out_h, cnt_out,
              pv, ev, sv, gix0, gix1, l0, l1, b0, b1, bounce, sem, sem1,
              agg_sh):
    s = lax.axis_index("s")
    lanes = lax.iota(jnp.int32, 16)
    ones16 = jnp.full((16,), 1.0, jnp.float32)
    zerof16 = jnp.zeros((16,), jnp.float32)

    pltpu.sync_copy(pv_h.at[s], pv)
    pltpu.sync_copy(et_h.at[s], ev)

    _zero_vmem(bounce, CPT, D)
    pltpu.sync_copy(bounce.at[pl.ds(0, CPT)], agg_sh.at[pl.ds(s * CPT, CPT)])
    _zero_vmem(b0, CB, D)
    plsc.subcore_barrier()

    def count_chunk(j, carry):
        for k in range(8):
            d16 = pv[j, pl.ds(k * 16, 16)] & 16383
            e16 = ev[j, pl.ds(k * 16, 16)]
            col = (d16 & 15) * 8 + e16
            plsc.store_scatter(b0, [lanes + k * 16, col], ones16)
            l0[0, pl.ds(k * 16, 16)] = lax.shift_right_logical(d16, 4)
        pltpu.sync_copy(b0, agg_sh.at[l0.at[0]], add=True)
        for k in range(8):
            d16 = pv[j, pl.ds(k * 16, 16)] & 16383
            e16 = ev[j, pl.ds(k * 16, 16)]
            col = (d16 & 15) * 8 + e16
            plsc.store_scatter(b0, [lanes + k * 16, col], zerof16)
        return carry
    lax.fori_loop(0, CH, count_chunk, 0)
    plsc.subcore_barrier()

    pltpu.sync_copy(agg_sh.at[pl.ds(s * CPT, CPT)], b0.at[pl.ds(0, CPT)])
    pltpu.sync_copy(b0.at[pl.ds(0, CPT)], cnt_out.at[pl.ds(s * CPT, CPT)])
    plsc.subcore_barrier()

    def scale_chunk(j, carry):
        for k in range(8):
            d16 = pv[j, pl.ds(k * 16, 16)] & 16383
            l0[0, pl.ds(k * 16, 16)] = lax.shift_right_logical(d16, 4)
        pltpu.sync_copy(cnt_out.at[l0.at[0]], b0)
        for k in range(8):
            d16 = pv[j, pl.ds(k * 16, 16)] & 16383
            e16 = ev[j, pl.ds(k * 16, 16)]
            cnt16 = plsc.load_gather(b0,
                                     [lanes + k * 16, (d16 & 15) * 8 + e16])
            sv[j, pl.ds(k * 16, 16)] = 1.0 / jnp.maximum(cnt16, 1.0)
        return carry
    lax.fori_loop(0, CH, scale_chunk, 0)

    _edge_agg(s, xw_h, out_h, pv, sv, gix0, gix1, l0, l1, b0, b1, bounce,
              sem, sem1, agg_sh)


def _agg_kernel():
    return pl.kernel(
        _agg_body,
        mesh=_sc_mesh(),
        compiler_params=pltpu.CompilerParams(needs_layout_passes=False),
        out_type=(jax.ShapeDtypeStruct((NP, D), jnp.float32),
                  jax.ShapeDtypeStruct((CROWS, D), jnp.float32)),
        scratch_types=[
            pltpu.VMEM((CH, CB), jnp.int32),
            pltpu.VMEM((CH, CB), jnp.int32),
            pltpu.VMEM((CH, CB), jnp.float32),
            pltpu.VMEM((1, CB), jnp.int32),
            pltpu.VMEM((1, CB), jnp.int32),
            pltpu.VMEM((1, CB), jnp.int32),
            pltpu.VMEM((1, CB), jnp.int32),
            pltpu.VMEM((CB, D), jnp.float32),
            pltpu.VMEM((CB, D), jnp.float32),
            pltpu.VMEM((HPB, D), jnp.float32),
            pltpu.SemaphoreType.DMA,
            pltpu.SemaphoreType.DMA,
            pltpu.VMEM_SHARED((NPH + 8, D), jnp.float32),
        ],
    )


def _xw_body(x_ref, w_ref, o_ref):
    o_ref[0] = jnp.dot(x_ref[...], w_ref[0],
                       preferred_element_type=jnp.float32)


def _per_rel_transform(x, w):
    return pl.pallas_call(
        _xw_body,
        grid=(NB, R),
        in_specs=[
            pl.BlockSpec((BN, D), lambda i, r: (i, 0)),
            pl.BlockSpec((1, D, D), lambda i, r: (r, 0, 0)),
        ],
        out_specs=pl.BlockSpec((1, BN, D), lambda i, r: (r, i, 0)),
        out_shape=jax.ShapeDtypeStruct((R, N, D), jnp.float32),
    )(x, w)


def _h_body(a_ref, x_ref, rt_ref, b_ref, o_ref):
    o_ref[...] = jnp.maximum(
        a_ref[...]
        + jnp.dot(x_ref[...], rt_ref[...], preferred_element_type=jnp.float32)
        + b_ref[...], 0.0)


def _layer_update(a, x, root, b):
    return pl.pallas_call(
        _h_body,
        grid=(NB,),
        in_specs=[
            pl.BlockSpec((BN, D), lambda i: (i, 0)),
            pl.BlockSpec((BN, D), lambda i: (i, 0)),
            pl.BlockSpec((D, D), lambda i: (0, 0)),
            pl.BlockSpec((1, D), lambda i: (0, 0)),
        ],
        out_specs=pl.BlockSpec((BN, D), lambda i: (i, 0)),
        out_shape=jax.ShapeDtypeStruct((N, D), jnp.float32),
    )(a, x, root, b.reshape(1, D))


def _fin_body(h_ref, wl_ref, bl_ref, o_ref):
    logits = jnp.dot(h_ref[...], wl_ref[...],
                     preferred_element_type=jnp.float32) + bl_ref[...]
    m = jnp.max(logits, axis=1, keepdims=True)
    ex = jnp.exp(logits - m)
    lse = jnp.log(jnp.sum(ex, axis=1, keepdims=True)) + m
    o_ref[...] = logits - lse


def _final_linear(h, wlin, blin):
    return pl.pallas_call(
        _fin_body,
        grid=(NB,),
        in_specs=[
            pl.BlockSpec((BN, D), lambda i: (i, 0)),
            pl.BlockSpec((D, DOUT), lambda i: (0, 0)),
            pl.BlockSpec((1, DOUT), lambda i: (0, 0)),
        ],
        out_specs=pl.BlockSpec((BN, DOUT), lambda i: (i, 0)),
        out_shape=jax.ShapeDtypeStruct((N, DOUT), jnp.float32),
    )(h, wlin, blin.reshape(1, DOUT))


def kernel(x, edge_index, edge_type, W1, root1, b1, W2, root2, b2, Wlin, blin):
    src = edge_index[0].astype(jnp.int32)
    dst = edge_index[1].astype(jnp.int32)
    et = edge_type.astype(jnp.int32)

    pad = EPAD - E
    gidx = et * N + src
    pv = gidx * 16384 + dst
    pv_h = jnp.concatenate(
        [pv, jnp.full((pad,), N, jnp.int32)]).reshape(NW, CH, CB)
    et_h = jnp.concatenate(
        [et, jnp.zeros((pad,), jnp.int32)]).reshape(NW, CH, CB)

    Ws = jnp.stack([W1, W2])
    roots = jnp.stack([root1, root2])
    bs = jnp.stack([b1, b2])

    def layer(h, wrb):
        W, root, b = wrb
        xw = _per_rel_transform(h, W).reshape(R * N, D)
        agg, _ = _agg_kernel()(xw, pv_h, et_h)
        return _layer_update(agg[:N], h, root, b), 0

    h2, _ = lax.scan(layer, x, (Ws, roots, bs))
    return _final_linear(h2, Wlin, blin)
